# baseline (speedup 1.0000x reference)
import functools

import jax
import jax.numpy as jnp
from jax import lax
from jax.experimental import pallas as pl
from jax.experimental.pallas import tpu as pltpu
from jax.experimental.pallas import tpu_sc as plsc

_D_MODEL = 1024
_DICT = 4096
_K = 8
_BLK_M = 128
_N_TOK = 4096

_NC = 2
_NS = 16
_NW = _NC * _NS
_TW = _N_TOK // _NW
_G = 4
_NG = _TW // _G
_LANES = 16


def _tc_body(x_ref, w_ref, valsb_ref, idx_ref, loss_ref):
    i = pl.program_id(0)
    coeffs = jax.lax.dot_general(
        x_ref[...], w_ref[...],
        dimension_numbers=(((1,), (1,)), ((), ())),
        preferred_element_type=jnp.float32,
    )
    colf = jax.lax.broadcasted_iota(
        jnp.int32, coeffs.shape, 1).astype(jnp.float32)
    work = coeffs
    neg = jnp.float32(-jnp.inf)
    big = jnp.float32(_DICT)
    ms, fs = [], []
    for _ in range(_K):
        m = jnp.max(work, axis=1, keepdims=True)
        eq = work == m
        first = jnp.min(jnp.where(eq, colf, big), axis=1, keepdims=True)
        work = jnp.where(eq, neg, work)
        ms.append(m)
        fs.append(first)
    valsb_ref[...] = jnp.concatenate(
        [jnp.broadcast_to(m, (_BLK_M, _LANES)) for m in ms], axis=1)
    idx_ref[...] = jnp.concatenate(fs, axis=1).astype(jnp.int32)

    @pl.when(i == 0)
    def _():
        loss_ref[0, 0] = jnp.float32(0.0)

    loss_ref[0, 0] += sum(jnp.sum(jnp.abs(m)) for m in ms)


def _tc_topk(xf, W_enc):
    grid = _N_TOK // _BLK_M
    return pl.pallas_call(
        _tc_body,
        grid=(grid,),
        in_specs=[
            pl.BlockSpec((_BLK_M, _D_MODEL), lambda i: (i, 0)),
            pl.BlockSpec((_DICT, _D_MODEL), lambda i: (0, 0)),
        ],
        out_specs=[
            pl.BlockSpec((_BLK_M, _K * _LANES), lambda i: (i, 0)),
            pl.BlockSpec((_BLK_M, _K), lambda i: (i, 0)),
            pl.BlockSpec(memory_space=pltpu.SMEM),
        ],
        out_shape=[
            jax.ShapeDtypeStruct((_N_TOK, _K * _LANES), jnp.float32),
            jax.ShapeDtypeStruct((_N_TOK, _K), jnp.int32),
            jax.ShapeDtypeStruct((1, 1), jnp.float32),
        ],
        compiler_params=pltpu.CompilerParams(
            dimension_semantics=("arbitrary",),
        ),
    )(xf, W_enc)


def _sc_decode_body(dict_ref, idx_ref, vals_ref, out_ref,
                    idx_v0, idx_v1, rows_v0, rows_v1, vals_v, out_v,
                    sem0, sem1):
    idx_v = (idx_v0, idx_v1)
    rows_v = (rows_v0, rows_v1)
    sems = (sem0, sem1)
    wid = lax.axis_index("s") * _NC + lax.axis_index("c")
    tok0 = wid * _TW

    def fire(g, b):
        off = pl.multiple_of((tok0 + g * _G) * _K, _G * _K)
        pltpu.sync_copy(idx_ref.at[pl.ds(off, _G * _K)], idx_v[b])
        pltpu.async_copy(dict_ref.at[idx_v[b]], rows_v[b], sems[b])

    def wait(b):
        pltpu.make_async_copy(
            dict_ref.at[pl.ds(0, _G * _K)], rows_v[b], sems[b]).wait()

    def compute(g, b):
        tok = tok0 + g * _G
        pltpu.sync_copy(vals_ref.at[pl.ds(tok, _G)], vals_v)
        rv = rows_v[b]
        for t in range(_G):
            splats = [vals_v[t, pl.ds(k * _LANES, _LANES)] for k in range(_K)]

            def dloop(d, _, t=t, splats=splats):
                sl = pl.ds(d * _LANES, _LANES)
                acc = splats[0] * rv[t * _K + 0, sl]
                for k in range(1, _K):
                    acc = acc + splats[k] * rv[t * _K + k, sl]
                out_v[t, sl] = acc
                return 0

            lax.fori_loop(0, _D_MODEL // _LANES, dloop, 0)
        pltpu.sync_copy(out_v, out_ref.at[pl.ds(tok, _G)])

    fire(0, 0)
    fire(1, 1)

    def pair(it, _):
        for b in range(2):
            g = 2 * it + b
            wait(b)
            compute(g, b)

            @pl.when(g + 2 < _NG)
            def _():
                fire(g + 2, b)
        return 0

    lax.fori_loop(0, _NG // 2, pair, 0)


@functools.cache
def _get_sc_decode():
    return pl.kernel(
        _sc_decode_body,
        out_type=jax.ShapeDtypeStruct((_N_TOK, _D_MODEL), jnp.float32),
        mesh=plsc.VectorSubcoreMesh(core_axis_name="c", subcore_axis_name="s"),
        scratch_types=[
            pltpu.VMEM((_G * _K,), jnp.int32),
            pltpu.VMEM((_G * _K,), jnp.int32),
            pltpu.VMEM((_G * _K, _D_MODEL), jnp.float32),
            pltpu.VMEM((_G * _K, _D_MODEL), jnp.float32),
            pltpu.VMEM((_G, _K * _LANES), jnp.float32),
            pltpu.VMEM((_G, _D_MODEL), jnp.float32),
            pltpu.SemaphoreType.DMA,
            pltpu.SemaphoreType.DMA,
        ],
    )


@jax.jit
def kernel(x, W_enc, dictionary):
    B, T, D = x.shape
    xf = x.reshape(_N_TOK, D)
    valsb, idx, loss = _tc_topk(xf, W_enc)
    idx_flat = idx.reshape(-1)
    offset = _get_sc_decode()(dictionary, idx_flat, valsb)
    sparsity_loss = loss[0, 0] / jnp.float32(_N_TOK * _DICT)
    return (offset.reshape(B, T, D), sparsity_loss)


# speedup vs baseline: 1.1089x; 1.1089x over previous
import functools

import jax
import jax.numpy as jnp
from jax import lax
from jax.experimental import pallas as pl
from jax.experimental.pallas import tpu as pltpu
from jax.experimental.pallas import tpu_sc as plsc

_D_MODEL = 1024
_DICT = 4096
_K = 8
_BLK_M = 128
_N_TOK = 4096

_NC = 2
_NS = 16
_NW = _NC * _NS
_TW = _N_TOK // _NW
_G = 4
_NG = _TW // _G
_LANES = 16


def _tc_body(x_ref, w_ref, valsb_ref, idx_ref, loss_ref):
    i = pl.program_id(0)
    coeffs = jax.lax.dot_general(
        x_ref[...], w_ref[...],
        dimension_numbers=(((1,), (1,)), ((), ())),
        preferred_element_type=jnp.float32,
    )
    colf = jax.lax.broadcasted_iota(
        jnp.int32, coeffs.shape, 1).astype(jnp.float32)
    work = coeffs
    neg = jnp.float32(-jnp.inf)
    big = jnp.float32(_DICT)
    ms, fs = [], []
    for _ in range(_K):
        m = jnp.max(work, axis=1, keepdims=True)
        eq = work == m
        first = jnp.min(jnp.where(eq, colf, big), axis=1, keepdims=True)
        work = jnp.where(eq, neg, work)
        ms.append(m)
        fs.append(first)
    valsb_ref[...] = jnp.concatenate(
        [jnp.broadcast_to(m, (_BLK_M, _LANES)) for m in ms], axis=1)
    idx_ref[...] = jnp.concatenate(fs, axis=1).astype(jnp.int32)

    @pl.when(i == 0)
    def _():
        loss_ref[0, 0] = jnp.float32(0.0)

    loss_ref[0, 0] += sum(jnp.sum(jnp.abs(m)) for m in ms)


def _tc_topk(xf, W_enc):
    grid = _N_TOK // _BLK_M
    return pl.pallas_call(
        _tc_body,
        grid=(grid,),
        in_specs=[
            pl.BlockSpec((_BLK_M, _D_MODEL), lambda i: (i, 0)),
            pl.BlockSpec((_DICT, _D_MODEL), lambda i: (0, 0)),
        ],
        out_specs=[
            pl.BlockSpec((_BLK_M, _K * _LANES), lambda i: (i, 0)),
            pl.BlockSpec((_BLK_M, _K), lambda i: (i, 0)),
            pl.BlockSpec(memory_space=pltpu.SMEM),
        ],
        out_shape=[
            jax.ShapeDtypeStruct((_N_TOK, _K * _LANES), jnp.float32),
            jax.ShapeDtypeStruct((_N_TOK, _K), jnp.int32),
            jax.ShapeDtypeStruct((1, 1), jnp.float32),
        ],
        compiler_params=pltpu.CompilerParams(
            dimension_semantics=("arbitrary",),
        ),
    )(xf, W_enc)


def _sc_decode_body(dict_ref, idx_ref, vals_ref, out_ref,
                    idx_v, vals_v, rows_v0, rows_v1, out_v0, out_v1,
                    gsem0, gsem1, osem0, osem1):
    rows_v = (rows_v0, rows_v1)
    out_v = (out_v0, out_v1)
    gsems = (gsem0, gsem1)
    osems = (osem0, osem1)
    wid = lax.axis_index("s") * _NC + lax.axis_index("c")
    tok0 = wid * _TW

    pltpu.sync_copy(idx_ref.at[pl.ds(tok0 * _K, _TW * _K)], idx_v)
    pltpu.sync_copy(vals_ref.at[pl.ds(tok0, _TW)], vals_v)

    def fire(g, b):
        off = pl.multiple_of(g * _G * _K, _G * _K)
        pltpu.async_copy(
            dict_ref.at[idx_v.at[pl.ds(off, _G * _K)]], rows_v[b], gsems[b])

    def wait_gather(b):
        pltpu.make_async_copy(
            dict_ref.at[pl.ds(0, _G * _K)], rows_v[b], gsems[b]).wait()

    def drain_store(b):
        pltpu.make_async_copy(
            out_v[b], out_ref.at[pl.ds(tok0, _G)], osems[b]).wait()

    fire(0, 0)
    fire(1, 1)

    def pair(it, _):
        for b in range(2):
            g = 2 * it + b
            wait_gather(b)

            @pl.when(g >= 2)
            def _():
                drain_store(b)

            rv = rows_v[b]
            ov = out_v[b]
            for t in range(_G):
                lt = g * _G + t
                splats = [vals_v[lt, pl.ds(k * _LANES, _LANES)]
                          for k in range(_K)]

                def dloop(d, _, t=t, splats=splats, rv=rv, ov=ov):
                    sl = pl.ds(d * _LANES, _LANES)
                    acc = splats[0] * rv[t * _K + 0, sl]
                    for k in range(1, _K):
                        acc = acc + splats[k] * rv[t * _K + k, sl]
                    ov[t, sl] = acc
                    return 0

                lax.fori_loop(0, _D_MODEL // _LANES, dloop, 0)
            pltpu.async_copy(
                ov, out_ref.at[pl.ds(tok0 + g * _G, _G)], osems[b])

            @pl.when(g + 2 < _NG)
            def _():
                fire(g + 2, b)
        return 0

    lax.fori_loop(0, _NG // 2, pair, 0)
    drain_store(0)
    drain_store(1)


@functools.cache
def _get_sc_decode():
    return pl.kernel(
        _sc_decode_body,
        out_type=jax.ShapeDtypeStruct((_N_TOK, _D_MODEL), jnp.float32),
        mesh=plsc.VectorSubcoreMesh(core_axis_name="c", subcore_axis_name="s"),
        scratch_types=[
            pltpu.VMEM((_TW * _K,), jnp.int32),
            pltpu.VMEM((_TW, _K * _LANES), jnp.float32),
            pltpu.VMEM((_G * _K, _D_MODEL), jnp.float32),
            pltpu.VMEM((_G * _K, _D_MODEL), jnp.float32),
            pltpu.VMEM((_G, _D_MODEL), jnp.float32),
            pltpu.VMEM((_G, _D_MODEL), jnp.float32),
            pltpu.SemaphoreType.DMA,
            pltpu.SemaphoreType.DMA,
            pltpu.SemaphoreType.DMA,
            pltpu.SemaphoreType.DMA,
        ],
    )


@jax.jit
def kernel(x, W_enc, dictionary):
    B, T, D = x.shape
    xf = x.reshape(_N_TOK, D)
    valsb, idx, loss = _tc_topk(xf, W_enc)
    idx_flat = idx.reshape(-1)
    offset = _get_sc_decode()(dictionary, idx_flat, valsb)
    sparsity_loss = loss[0, 0] / jnp.float32(_N_TOK * _DICT)
    return (offset.reshape(B, T, D), sparsity_loss)


# speedup vs baseline: 1.1532x; 1.0399x over previous
import functools

import jax
import jax.numpy as jnp
from jax import lax
from jax.experimental import pallas as pl
from jax.experimental.pallas import tpu as pltpu
from jax.experimental.pallas import tpu_sc as plsc

_D_MODEL = 1024
_DICT = 4096
_K = 8
_BLK_M = 128
_N_TOK = 4096

_NC = 2
_NS = 16
_NW = _NC * _NS
_TW = _N_TOK // _NW
_G = 4
_NG = _TW // _G
_LANES = 16


def _tc_body(x_ref, w_ref, valsb_ref, idx_ref, loss_ref):
    i = pl.program_id(0)
    coeffs = jax.lax.dot_general(
        x_ref[...], w_ref[...],
        dimension_numbers=(((1,), (1,)), ((), ())),
        preferred_element_type=jnp.float32,
    )
    colf = jax.lax.broadcasted_iota(
        jnp.int32, coeffs.shape, 1).astype(jnp.float32)
    work = coeffs
    neg = jnp.float32(-jnp.inf)
    big = jnp.float32(_DICT)
    ms, fs = [], []
    for _ in range(_K):
        m = jnp.max(work, axis=1, keepdims=True)
        eq = work == m
        first = jnp.min(jnp.where(eq, colf, big), axis=1, keepdims=True)
        work = jnp.where(eq, neg, work)
        ms.append(m)
        fs.append(first)
    valsb_ref[...] = jnp.concatenate(
        [jnp.broadcast_to(m, (_BLK_M, _LANES)) for m in ms], axis=1)
    idx_ref[...] = jnp.concatenate(fs, axis=1).astype(jnp.int32)

    @pl.when(i == 0)
    def _():
        loss_ref[0, 0] = jnp.float32(0.0)

    loss_ref[0, 0] += sum(jnp.sum(jnp.abs(m)) for m in ms)


def _tc_topk(xf, W_enc):
    n_tok = xf.shape[0]
    grid = n_tok // _BLK_M
    return pl.pallas_call(
        _tc_body,
        grid=(grid,),
        in_specs=[
            pl.BlockSpec((_BLK_M, _D_MODEL), lambda i: (i, 0)),
            pl.BlockSpec((_DICT, _D_MODEL), lambda i: (0, 0)),
        ],
        out_specs=[
            pl.BlockSpec((_BLK_M, _K * _LANES), lambda i: (i, 0)),
            pl.BlockSpec((_BLK_M, _K), lambda i: (i, 0)),
            pl.BlockSpec(memory_space=pltpu.SMEM),
        ],
        out_shape=[
            jax.ShapeDtypeStruct((n_tok, _K * _LANES), jnp.float32),
            jax.ShapeDtypeStruct((n_tok, _K), jnp.int32),
            jax.ShapeDtypeStruct((1, 1), jnp.float32),
        ],
        compiler_params=pltpu.CompilerParams(
            dimension_semantics=("arbitrary",),
        ),
    )(xf, W_enc)


def _sc_decode_body(tw, dict_ref, idx_ref, vals_ref, out_ref,
                    idx_v, vals_v, rows_v0, rows_v1, out_v0, out_v1,
                    gsem0, gsem1, osem0, osem1):
    ng = tw // _G
    rows_v = (rows_v0, rows_v1)
    out_v = (out_v0, out_v1)
    gsems = (gsem0, gsem1)
    osems = (osem0, osem1)
    wid = lax.axis_index("s") * _NC + lax.axis_index("c")
    tok0 = wid * tw

    pltpu.sync_copy(idx_ref.at[pl.ds(tok0 * _K, tw * _K)], idx_v)
    pltpu.sync_copy(vals_ref.at[pl.ds(tok0, tw)], vals_v)

    def fire(g, b):
        off = pl.multiple_of(g * _G * _K, _G * _K)
        pltpu.async_copy(
            dict_ref.at[idx_v.at[pl.ds(off, _G * _K)]], rows_v[b], gsems[b])

    def wait_gather(b):
        pltpu.make_async_copy(
            dict_ref.at[pl.ds(0, _G * _K)], rows_v[b], gsems[b]).wait()

    def drain_store(b):
        pltpu.make_async_copy(
            out_v[b], out_ref.at[pl.ds(tok0, _G)], osems[b]).wait()

    fire(0, 0)
    fire(1, 1)

    def pair(it, _):
        for b in range(2):
            g = 2 * it + b
            wait_gather(b)

            @pl.when(g >= 2)
            def _():
                drain_store(b)

            rv = rows_v[b]
            ov = out_v[b]
            for t in range(_G):
                lt = g * _G + t
                splats = [vals_v[lt, pl.ds(k * _LANES, _LANES)]
                          for k in range(_K)]

                def dloop(d, _, t=t, splats=splats, rv=rv, ov=ov):
                    sl = pl.ds(d * _LANES, _LANES)
                    acc = splats[0] * rv[t * _K + 0, sl]
                    for k in range(1, _K):
                        acc = acc + splats[k] * rv[t * _K + k, sl]
                    ov[t, sl] = acc
                    return 0

                lax.fori_loop(0, _D_MODEL // _LANES, dloop, 0)
            pltpu.async_copy(
                ov, out_ref.at[pl.ds(tok0 + g * _G, _G)], osems[b])

            @pl.when(g + 2 < ng)
            def _():
                fire(g + 2, b)
        return 0

    lax.fori_loop(0, ng // 2, pair, 0)
    drain_store(0)
    drain_store(1)


@functools.cache
def _get_sc_decode(n_tok):
    tw = n_tok // _NW
    return pl.kernel(
        functools.partial(_sc_decode_body, tw),
        out_type=jax.ShapeDtypeStruct((n_tok, _D_MODEL), jnp.float32),
        mesh=plsc.VectorSubcoreMesh(core_axis_name="c", subcore_axis_name="s"),
        scratch_types=[
            pltpu.VMEM((tw * _K,), jnp.int32),
            pltpu.VMEM((tw, _K * _LANES), jnp.float32),
            pltpu.VMEM((_G * _K, _D_MODEL), jnp.float32),
            pltpu.VMEM((_G * _K, _D_MODEL), jnp.float32),
            pltpu.VMEM((_G, _D_MODEL), jnp.float32),
            pltpu.VMEM((_G, _D_MODEL), jnp.float32),
            pltpu.SemaphoreType.DMA,
            pltpu.SemaphoreType.DMA,
            pltpu.SemaphoreType.DMA,
            pltpu.SemaphoreType.DMA,
        ],
    )


_N_CHUNKS = 2


@jax.jit
def kernel(x, W_enc, dictionary):
    B, T, D = x.shape
    xf = x.reshape(_N_TOK, D)
    chunk = _N_TOK // _N_CHUNKS
    offs, losses = [], []
    for c in range(_N_CHUNKS):
        valsb, idx, loss = _tc_topk(
            lax.slice_in_dim(xf, c * chunk, (c + 1) * chunk), W_enc)
        off = _get_sc_decode(chunk)(dictionary, idx.reshape(-1), valsb)
        offs.append(off)
        losses.append(loss[0, 0])
    offset = jnp.concatenate(offs, axis=0)
    sparsity_loss = sum(losses) / jnp.float32(_N_TOK * _DICT)
    return (offset.reshape(B, T, D), sparsity_loss)


# speedup vs baseline: 1.1574x; 1.0036x over previous
import functools

import jax
import jax.numpy as jnp
from jax import lax
from jax.experimental import pallas as pl
from jax.experimental.pallas import tpu as pltpu
from jax.experimental.pallas import tpu_sc as plsc

_D_MODEL = 1024
_DICT = 4096
_K = 8
_BLK_M = 128
_N_TOK = 4096

_NC = 2
_NS = 16
_NW = _NC * _NS
_TW = _N_TOK // _NW
_G = 4
_NG = _TW // _G
_LANES = 16


def _tc_body(x_ref, w_ref, valsb_ref, idx_ref, loss_ref):
    i = pl.program_id(0)
    coeffs = jax.lax.dot_general(
        x_ref[...], w_ref[...],
        dimension_numbers=(((1,), (1,)), ((), ())),
        preferred_element_type=jnp.float32,
    )
    colf = jax.lax.broadcasted_iota(
        jnp.int32, coeffs.shape, 1).astype(jnp.float32)
    work = coeffs
    neg = jnp.float32(-jnp.inf)
    big = jnp.float32(_DICT)
    ms, fs = [], []
    for _ in range(_K):
        m = jnp.max(work, axis=1, keepdims=True)
        eq = work == m
        first = jnp.min(jnp.where(eq, colf, big), axis=1, keepdims=True)
        work = jnp.where(eq, neg, work)
        ms.append(m)
        fs.append(first)
    valsb_ref[...] = jnp.concatenate(
        [jnp.broadcast_to(m, (_BLK_M, _LANES)) for m in ms], axis=1)
    idx_ref[...] = jnp.concatenate(fs, axis=1).astype(jnp.int32)

    @pl.when(i == 0)
    def _():
        loss_ref[0, 0] = jnp.float32(0.0)

    loss_ref[0, 0] += sum(jnp.sum(jnp.abs(m)) for m in ms)


def _tc_topk(xf, W_enc):
    n_tok = xf.shape[0]
    grid = n_tok // _BLK_M
    return pl.pallas_call(
        _tc_body,
        grid=(grid,),
        in_specs=[
            pl.BlockSpec((_BLK_M, _D_MODEL), lambda i: (i, 0)),
            pl.BlockSpec((_DICT, _D_MODEL), lambda i: (0, 0)),
        ],
        out_specs=[
            pl.BlockSpec((_BLK_M, _K * _LANES), lambda i: (i, 0)),
            pl.BlockSpec((_BLK_M, _K), lambda i: (i, 0)),
            pl.BlockSpec(memory_space=pltpu.SMEM),
        ],
        out_shape=[
            jax.ShapeDtypeStruct((n_tok, _K * _LANES), jnp.float32),
            jax.ShapeDtypeStruct((n_tok, _K), jnp.int32),
            jax.ShapeDtypeStruct((1, 1), jnp.float32),
        ],
        compiler_params=pltpu.CompilerParams(
            dimension_semantics=("arbitrary",),
        ),
    )(xf, W_enc)


def _sc_decode_body(tw, dict_ref, idx_ref, vals_ref, out_ref,
                    idx_v, vals_v, rows_v0, rows_v1, out_v0, out_v1,
                    gsem0, gsem1, osem0, osem1):
    ng = tw // _G
    rows_v = (rows_v0, rows_v1)
    out_v = (out_v0, out_v1)
    gsems = (gsem0, gsem1)
    osems = (osem0, osem1)
    wid = lax.axis_index("s") * _NC + lax.axis_index("c")
    tok0 = wid * tw

    pltpu.sync_copy(idx_ref.at[pl.ds(tok0 * _K, tw * _K)], idx_v)
    pltpu.sync_copy(vals_ref.at[pl.ds(tok0, tw)], vals_v)

    def fire(g, b):
        off = pl.multiple_of(g * _G * _K, _G * _K)
        pltpu.async_copy(
            dict_ref.at[idx_v.at[pl.ds(off, _G * _K)]], rows_v[b], gsems[b])

    def wait_gather(b):
        pltpu.make_async_copy(
            dict_ref.at[pl.ds(0, _G * _K)], rows_v[b], gsems[b]).wait()

    def drain_store(b):
        pltpu.make_async_copy(
            out_v[b], out_ref.at[pl.ds(tok0, _G)], osems[b]).wait()

    fire(0, 0)
    fire(1, 1)

    def pair(it, _):
        for b in range(2):
            g = 2 * it + b
            wait_gather(b)

            @pl.when(g >= 2)
            def _():
                drain_store(b)

            rv = rows_v[b]
            ov = out_v[b]
            for t in range(_G):
                lt = g * _G + t
                splats = [vals_v[lt, pl.ds(k * _LANES, _LANES)]
                          for k in range(_K)]

                def dloop(d, _, t=t, splats=splats, rv=rv, ov=ov):
                    sl = pl.ds(d * _LANES, _LANES)
                    acc = splats[0] * rv[t * _K + 0, sl]
                    for k in range(1, _K):
                        acc = acc + splats[k] * rv[t * _K + k, sl]
                    ov[t, sl] = acc
                    return 0

                lax.fori_loop(0, _D_MODEL // _LANES, dloop, 0)
            pltpu.async_copy(
                ov, out_ref.at[pl.ds(tok0 + g * _G, _G)], osems[b])

            @pl.when(g + 2 < ng)
            def _():
                fire(g + 2, b)
        return 0

    lax.fori_loop(0, ng // 2, pair, 0)
    drain_store(0)
    drain_store(1)


@functools.cache
def _get_sc_decode(n_tok):
    tw = n_tok // _NW
    return pl.kernel(
        functools.partial(_sc_decode_body, tw),
        out_type=jax.ShapeDtypeStruct((n_tok, _D_MODEL), jnp.float32),
        mesh=plsc.VectorSubcoreMesh(core_axis_name="c", subcore_axis_name="s"),
        scratch_types=[
            pltpu.VMEM((tw * _K,), jnp.int32),
            pltpu.VMEM((tw, _K * _LANES), jnp.float32),
            pltpu.VMEM((_G * _K, _D_MODEL), jnp.float32),
            pltpu.VMEM((_G * _K, _D_MODEL), jnp.float32),
            pltpu.VMEM((_G, _D_MODEL), jnp.float32),
            pltpu.VMEM((_G, _D_MODEL), jnp.float32),
            pltpu.SemaphoreType.DMA,
            pltpu.SemaphoreType.DMA,
            pltpu.SemaphoreType.DMA,
            pltpu.SemaphoreType.DMA,
        ],
    )


_N_CHUNKS = 2


@jax.jit
def kernel(x, W_enc, dictionary):
    B, T, D = x.shape
    xf = x.reshape(_N_TOK, D)
    chunk = _N_TOK // _N_CHUNKS
    offs, losses, parts = [], [], []
    for c in range(_N_CHUNKS):
        valsb, idx, loss = _tc_topk(
            lax.slice_in_dim(xf, c * chunk, (c + 1) * chunk), W_enc)
        parts.append((valsb, idx))
        losses.append(loss[0, 0])
    for valsb, idx in parts:
        offs.append(_get_sc_decode(chunk)(dictionary, idx.reshape(-1), valsb))
    offset = jnp.concatenate(offs, axis=0)
    sparsity_loss = sum(losses) / jnp.float32(_N_TOK * _DICT)
    return (offset.reshape(B, T, D), sparsity_loss)


# speedup vs baseline: 1.2389x; 1.0705x over previous
import functools

import jax
import jax.numpy as jnp
from jax import lax
from jax.experimental import pallas as pl
from jax.experimental.pallas import tpu as pltpu
from jax.experimental.pallas import tpu_sc as plsc

_D_MODEL = 1024
_DICT = 4096
_K = 8
_BLK_M = 128
_N_TOK = 4096

_NC = 2
_NS = 16
_NW = _NC * _NS
_TW = _N_TOK // _NW
_G = 4
_NG = _TW // _G
_LANES = 16


def _tc_body(x_ref, w_ref, valsb_ref, idx_ref, loss_ref):
    i = pl.program_id(0)
    coeffs = jax.lax.dot_general(
        x_ref[...], w_ref[...],
        dimension_numbers=(((1,), (1,)), ((), ())),
        preferred_element_type=jnp.float32,
    )
    colf = jax.lax.broadcasted_iota(
        jnp.int32, coeffs.shape, 1).astype(jnp.float32)
    work = coeffs
    neg = jnp.float32(-jnp.inf)
    big = jnp.float32(_DICT)
    ms, fs = [], []
    for _ in range(_K):
        m = jnp.max(work, axis=1, keepdims=True)
        eq = work == m
        first = jnp.min(jnp.where(eq, colf, big), axis=1, keepdims=True)
        work = jnp.where(eq, neg, work)
        ms.append(m)
        fs.append(first)
    valsb_ref[...] = jnp.concatenate(
        [jnp.broadcast_to(m, (_BLK_M, _LANES)) for m in ms], axis=1)
    idx_ref[...] = jnp.concatenate(fs, axis=1).astype(jnp.int32)

    @pl.when(i == 0)
    def _():
        loss_ref[0, 0] = jnp.float32(0.0)

    loss_ref[0, 0] += sum(jnp.sum(jnp.abs(m)) for m in ms)


def _tc_topk(xf, W_enc):
    n_tok = xf.shape[0]
    grid = n_tok // _BLK_M
    return pl.pallas_call(
        _tc_body,
        grid=(grid,),
        in_specs=[
            pl.BlockSpec((_BLK_M, _D_MODEL), lambda i: (i, 0)),
            pl.BlockSpec((_DICT, _D_MODEL), lambda i: (0, 0)),
        ],
        out_specs=[
            pl.BlockSpec((_BLK_M, _K * _LANES), lambda i: (i, 0)),
            pl.BlockSpec((_BLK_M, _K), lambda i: (i, 0)),
            pl.BlockSpec(memory_space=pltpu.SMEM),
        ],
        out_shape=[
            jax.ShapeDtypeStruct((n_tok, _K * _LANES), jnp.float32),
            jax.ShapeDtypeStruct((n_tok, _K), jnp.int32),
            jax.ShapeDtypeStruct((1, 1), jnp.float32),
        ],
        compiler_params=pltpu.CompilerParams(
            dimension_semantics=("arbitrary",),
        ),
    )(xf, W_enc)


def _sc_decode_body(tw, dict_ref, idx_ref, vals_ref, out_ref,
                    idx_v, vals_v, rows_v0, rows_v1, out_v0, out_v1,
                    gsem0, gsem1, osem0, osem1):
    ng = tw // _G
    rows_v = (rows_v0, rows_v1)
    out_v = (out_v0, out_v1)
    gsems = (gsem0, gsem1)
    osems = (osem0, osem1)
    wid = lax.axis_index("s") * _NC + lax.axis_index("c")
    tok0 = wid * tw

    pltpu.sync_copy(idx_ref.at[pl.ds(tok0 * _K, tw * _K)], idx_v)
    pltpu.sync_copy(vals_ref.at[pl.ds(tok0, tw)], vals_v)

    def fire(g, b):
        off = pl.multiple_of(g * _G * _K, _G * _K)
        pltpu.async_copy(
            dict_ref.at[idx_v.at[pl.ds(off, _G * _K)]], rows_v[b], gsems[b])

    def wait_gather(b):
        pltpu.make_async_copy(
            dict_ref.at[pl.ds(0, _G * _K)], rows_v[b], gsems[b]).wait()

    def drain_store(b):
        pltpu.make_async_copy(
            out_v[b], out_ref.at[pl.ds(tok0, _G)], osems[b]).wait()

    fire(0, 0)
    fire(1, 1)

    def pair(it, _):
        for b in range(2):
            g = 2 * it + b
            wait_gather(b)

            @pl.when(g >= 2)
            def _():
                drain_store(b)

            rv = rows_v[b]
            ov = out_v[b]
            for t in range(_G):
                lt = g * _G + t
                splats = [vals_v[lt, pl.ds(k * _LANES, _LANES)]
                          for k in range(_K)]

                @plsc.parallel_loop(0, _D_MODEL // _LANES, unroll=4)
                def dloop(d, t=t, splats=splats, rv=rv, ov=ov):
                    sl = pl.ds(d * _LANES, _LANES)
                    acc = splats[0] * rv[t * _K + 0, sl]
                    for k in range(1, _K):
                        acc = acc + splats[k] * rv[t * _K + k, sl]
                    ov[t, sl] = acc
            pltpu.async_copy(
                ov, out_ref.at[pl.ds(tok0 + g * _G, _G)], osems[b])

            @pl.when(g + 2 < ng)
            def _():
                fire(g + 2, b)
        return 0

    lax.fori_loop(0, ng // 2, pair, 0)
    drain_store(0)
    drain_store(1)


@functools.cache
def _get_sc_decode(n_tok):
    tw = n_tok // _NW
    return pl.kernel(
        functools.partial(_sc_decode_body, tw),
        out_type=jax.ShapeDtypeStruct((n_tok, _D_MODEL), jnp.float32),
        mesh=plsc.VectorSubcoreMesh(core_axis_name="c", subcore_axis_name="s"),
        scratch_types=[
            pltpu.VMEM((tw * _K,), jnp.int32),
            pltpu.VMEM((tw, _K * _LANES), jnp.float32),
            pltpu.VMEM((_G * _K, _D_MODEL), jnp.float32),
            pltpu.VMEM((_G * _K, _D_MODEL), jnp.float32),
            pltpu.VMEM((_G, _D_MODEL), jnp.float32),
            pltpu.VMEM((_G, _D_MODEL), jnp.float32),
            pltpu.SemaphoreType.DMA,
            pltpu.SemaphoreType.DMA,
            pltpu.SemaphoreType.DMA,
            pltpu.SemaphoreType.DMA,
        ],
    )


_N_CHUNKS = 2


@jax.jit
def kernel(x, W_enc, dictionary):
    B, T, D = x.shape
    xf = x.reshape(_N_TOK, D)
    chunk = _N_TOK // _N_CHUNKS
    offs, losses, parts = [], [], []
    for c in range(_N_CHUNKS):
        valsb, idx, loss = _tc_topk(
            lax.slice_in_dim(xf, c * chunk, (c + 1) * chunk), W_enc)
        parts.append((valsb, idx))
        losses.append(loss[0, 0])
    for valsb, idx in parts:
        offs.append(_get_sc_decode(chunk)(dictionary, idx.reshape(-1), valsb))
    offset = jnp.concatenate(offs, axis=0)
    sparsity_loss = sum(losses) / jnp.float32(_N_TOK * _DICT)
    return (offset.reshape(B, T, D), sparsity_loss)
